# Initial kernel scaffold; baseline (speedup 1.0000x reference)
import jax
import jax.numpy as jnp
from jax.experimental import pallas as pl


def kernel(holder, table):
    raise NotImplementedError("write your pallas kernel here")


# speedup vs baseline: 1.5589x; 1.5589x over previous
import functools

import jax
import jax.numpy as jnp
from jax import lax
from jax.experimental import pallas as pl
from jax.experimental.pallas import tpu as pltpu
from jax.experimental.pallas import tpu_sc as plsc

_EMB = 32
_NUM_CORES = 2
_NUM_SUBCORES = 16
_NW = _NUM_CORES * _NUM_SUBCORES


def _make_gather(n, emb, chunk):
    assert n % _NW == 0
    per_w = n // _NW
    assert per_w % chunk == 0
    nchunk = per_w // chunk
    mesh = plsc.VectorSubcoreMesh(core_axis_name="c", subcore_axis_name="s")

    @functools.partial(
        pl.kernel,
        mesh=mesh,
        out_type=jax.ShapeDtypeStruct((n, emb), jnp.float32),
        scratch_types=[
            pltpu.VMEM((per_w,), jnp.int32),
            pltpu.VMEM((chunk, emb), jnp.float32),
            pltpu.SemaphoreType.DMA,
        ],
        compiler_params=pltpu.CompilerParams(use_tc_tiling_on_sc=False),
    )
    def emb_kernel(idx_hbm, table_hbm, out_hbm, idx_v, rows_v, gsem):
        wid = lax.axis_index("s") * _NUM_CORES + lax.axis_index("c")
        base = wid * per_w
        pltpu.sync_copy(idx_hbm.at[pl.ds(base, per_w)], idx_v)
        for c in range(nchunk):
            off = c * chunk
            pltpu.async_copy(
                table_hbm.at[idx_v.at[pl.ds(off, chunk)]], rows_v, gsem
            ).wait()
            pltpu.sync_copy(rows_v, out_hbm.at[pl.ds(base + off, chunk)])

    return emb_kernel


def kernel(holder, table):
    b, f = holder.shape
    idx = holder.reshape(-1).astype(jnp.int32)
    out = _make_gather(b * f, _EMB, 1024)(idx, table)
    return out.reshape(b, f, _EMB)


# speedup vs baseline: 1.5753x; 1.0105x over previous
import functools

import jax
import jax.numpy as jnp
from jax import lax
from jax.experimental import pallas as pl
from jax.experimental.pallas import tpu as pltpu
from jax.experimental.pallas import tpu_sc as plsc

_EMB = 32
_NUM_CORES = 2
_NUM_SUBCORES = 16
_NW = _NUM_CORES * _NUM_SUBCORES


def _make_gather(n, emb, chunk, nbuf):
    assert n % _NW == 0
    per_w = n // _NW
    assert per_w % chunk == 0
    nchunk = per_w // chunk
    assert nchunk >= nbuf
    mesh = plsc.VectorSubcoreMesh(core_axis_name="c", subcore_axis_name="s")

    @functools.partial(
        pl.kernel,
        mesh=mesh,
        out_type=jax.ShapeDtypeStruct((n, emb), jnp.float32),
        scratch_types=[
            pltpu.VMEM((per_w,), jnp.int32),
            [pltpu.VMEM((chunk, emb), jnp.float32) for _ in range(nbuf)],
            [pltpu.SemaphoreType.DMA for _ in range(nbuf)],
            [pltpu.SemaphoreType.DMA for _ in range(nbuf)],
        ],
        compiler_params=pltpu.CompilerParams(use_tc_tiling_on_sc=False),
    )
    def emb_kernel(idx_hbm, table_hbm, out_hbm, idx_v, rows, gsems, ssems):
        wid = lax.axis_index("s") * _NUM_CORES + lax.axis_index("c")
        base = wid * per_w

        def gather(c):
            b = c % nbuf
            pltpu.async_copy(
                table_hbm.at[idx_v.at[pl.ds(c * chunk, chunk)]],
                rows[b],
                gsems[b],
            )

        pltpu.sync_copy(idx_hbm.at[pl.ds(base, per_w)], idx_v)
        for b in range(nbuf):
            gather(b)
        for c in range(nchunk):
            b = c % nbuf
            if c >= 1 and (c - 1) + nbuf < nchunk:
                bb = (c - 1) % nbuf
                pltpu.make_async_copy(
                    rows[bb],
                    out_hbm.at[pl.ds(base + (c - 1) * chunk, chunk)],
                    ssems[bb],
                ).wait()
                gather((c - 1) + nbuf)
            pltpu.make_async_copy(
                table_hbm.at[idx_v.at[pl.ds(c * chunk, chunk)]],
                rows[b],
                gsems[b],
            ).wait()
            pltpu.async_copy(
                rows[b], out_hbm.at[pl.ds(base + c * chunk, chunk)], ssems[b]
            )
        for c in range(nchunk - nbuf, nchunk):
            b = c % nbuf
            pltpu.make_async_copy(
                rows[b], out_hbm.at[pl.ds(base + c * chunk, chunk)], ssems[b]
            ).wait()

    return emb_kernel


def kernel(holder, table):
    b, f = holder.shape
    idx = holder.reshape(-1).astype(jnp.int32)
    out = _make_gather(b * f, _EMB, 832, 4)(idx, table)
    return out.reshape(b, f, _EMB)


# speedup vs baseline: 1.6736x; 1.0624x over previous
import functools

import jax
import jax.numpy as jnp
from jax import lax
from jax.experimental import pallas as pl
from jax.experimental.pallas import tpu as pltpu
from jax.experimental.pallas import tpu_sc as plsc

_EMB = 32
_NUM_CORES = 2
_NUM_SUBCORES = 16
_NW = _NUM_CORES * _NUM_SUBCORES


def _make_gather(n, emb, chunk, nbuf):
    assert n % _NW == 0
    per_w = n // _NW
    assert per_w % chunk == 0
    nchunk = per_w // chunk
    assert nchunk >= nbuf
    mesh = plsc.VectorSubcoreMesh(core_axis_name="c", subcore_axis_name="s")

    @functools.partial(
        pl.kernel,
        mesh=mesh,
        out_type=jax.ShapeDtypeStruct((n, emb), jnp.float32),
        scratch_types=[
            pltpu.VMEM((per_w,), jnp.int32),
            [pltpu.VMEM((chunk, emb), jnp.float32) for _ in range(nbuf)],
            [pltpu.SemaphoreType.DMA for _ in range(nbuf)],
            [pltpu.SemaphoreType.DMA for _ in range(nbuf)],
        ],
        compiler_params=pltpu.CompilerParams(use_tc_tiling_on_sc=False),
    )
    def emb_kernel(idx_hbm, table_hbm, out_hbm, idx_v, rows, gsems, ssems):
        wid = lax.axis_index("s") * _NUM_CORES + lax.axis_index("c")
        base = wid * per_w

        def gather(c):
            b = c % nbuf
            pltpu.async_copy(
                table_hbm.at[idx_v.at[pl.ds(c * chunk, chunk)]],
                rows[b],
                gsems[b],
            )

        pltpu.sync_copy(idx_hbm.at[pl.ds(base, per_w)], idx_v)
        for b in range(nbuf):
            gather(b)
        for c in range(nchunk):
            b = c % nbuf
            if c >= 1 and (c - 1) + nbuf < nchunk:
                bb = (c - 1) % nbuf
                pltpu.make_async_copy(
                    rows[bb],
                    out_hbm.at[pl.ds(base + (c - 1) * chunk, chunk)],
                    ssems[bb],
                ).wait()
                gather((c - 1) + nbuf)
            pltpu.make_async_copy(
                table_hbm.at[idx_v.at[pl.ds(c * chunk, chunk)]],
                rows[b],
                gsems[b],
            ).wait()
            pltpu.async_copy(
                rows[b], out_hbm.at[pl.ds(base + c * chunk, chunk)], ssems[b]
            )
        for c in range(nchunk - nbuf, nchunk):
            b = c % nbuf
            pltpu.make_async_copy(
                rows[b], out_hbm.at[pl.ds(base + c * chunk, chunk)], ssems[b]
            ).wait()

    return emb_kernel


def kernel(holder, table):
    b, f = holder.shape
    idx = holder.T.reshape(-1).astype(jnp.int32)
    out = _make_gather(b * f, _EMB, 832, 4)(idx, table)
    return out.reshape(f, b, _EMB).transpose(1, 0, 2)
